# baseline (speedup 1.0000x reference)
import functools

import jax
import jax.numpy as jnp
from jax import lax
from jax.experimental import pallas as pl
from jax.experimental.pallas import tpu as pltpu
from jax.experimental.pallas import tpu_sc as plsc

N = 10000
NPAD = 10240
E = 320000
D = 128

NC = 2
NS = 16
EDGES_PER_WORKER = E // (NC * NS)
BATCH = 80
NBATCH = EDGES_PER_WORKER // BATCH
SB = 25
NSUPER = NBATCH // SB
NRING = 3
ROWS_PER_TILE = NPAD // NS
ROW_CHUNK = BATCH
NCHUNK = ROWS_PER_TILE // ROW_CHUNK


def _matmul_body(x_ref, w_ref, o_ref):
    o_ref[...] = jnp.dot(x_ref[...], w_ref[...],
                         preferred_element_type=jnp.float32)


def _epilogue_body(p0_ref, p1_ref, prior_ref, b_ref, o_ref):
    o_ref[...] = prior_ref[...] * (p0_ref[0] + p1_ref[0]) + b_ref[...]


def _spmm_body(support_hbm, src_hbm, dst_hbm, w_hbm, out_hbm,
               src_t, dst_t, w_t, rows_t, sem_g, sem_s, acc_sh):
    c = lax.axis_index("c")
    s = lax.axis_index("s")
    row0 = s * ROWS_PER_TILE

    zeros16 = jnp.zeros((16,), jnp.float32)

    def _zero_row(r, carry):
        for k in range(D // 16):
            rows_t[0, r, pl.ds(k * 16, 16)] = zeros16
        return carry

    lax.fori_loop(0, ROW_CHUNK, _zero_row, 0)
    for t in range(NCHUNK):
        pltpu.sync_copy(rows_t.at[0], acc_sh.at[pl.ds(row0 + t * ROW_CHUNK,
                                                      ROW_CHUNK)])
    plsc.subcore_barrier()

    def _g_start(j, p):
        pltpu.async_copy(support_hbm.at[src_t.at[j]], rows_t.at[p],
                         sem_g.at[p])

    def _g_wait(j, p):
        pltpu.make_async_copy(support_hbm.at[src_t.at[j]], rows_t.at[p],
                              sem_g.at[p]).wait()

    def _s_start(j, p):
        pltpu.async_copy(rows_t.at[p], acc_sh.at[dst_t.at[j]],
                         sem_s.at[p], add=True)

    def _s_wait(p):
        pltpu.make_async_copy(rows_t.at[p], acc_sh.at[dst_t.at[0]],
                              sem_s.at[p]).wait()

    def _scale(j, p):
        for g in range(BATCH // 16):
            wv16 = w_t[j, pl.ds(g * 16, 16)]

            def _edge(e16, ecarry, wv16=wv16, g=g):
                wv = lax.gather(
                    wv16, jnp.full((16, 1), e16, jnp.int32),
                    dimension_numbers=lax.GatherDimensionNumbers(
                        offset_dims=(), collapsed_slice_dims=(0,),
                        start_index_map=(0,)),
                    slice_sizes=(1,),
                    mode=lax.GatherScatterMode.PROMISE_IN_BOUNDS)
                e = g * 16 + e16
                for k in range(D // 16):
                    sl = pl.ds(k * 16, 16)
                    rows_t[p, e, sl] = rows_t[p, e, sl] * wv
                return ecarry

            lax.fori_loop(0, 16, _edge, 0)

    def _super(t, carry):
        pltpu.sync_copy(src_hbm.at[c, s, t], src_t)
        pltpu.sync_copy(dst_hbm.at[c, s, t], dst_t)
        pltpu.sync_copy(w_hbm.at[c, s, t], w_t)

        _g_start(0, 0)
        _g_start(1, 1)

        def _triple(k, kcarry):
            base = 3 * k
            _g_wait(base, 0)
            _scale(base, 0)
            _s_start(base, 0)

            @pl.when(k >= 1)
            def _():
                _s_wait(2)

            _g_start(base + 2, 2)

            _g_wait(base + 1, 1)
            _scale(base + 1, 1)
            _s_start(base + 1, 1)
            _s_wait(0)
            _g_start(base + 3, 0)

            _g_wait(base + 2, 2)
            _scale(base + 2, 2)
            _s_start(base + 2, 2)

            @pl.when(k <= 6)
            def _():
                _s_wait(1)
                _g_start(base + 4, 1)

            return kcarry

        lax.fori_loop(0, SB // 3, _triple, 0)

        _g_wait(SB - 1, 0)
        _scale(SB - 1, 0)
        _s_start(SB - 1, 0)

        for q in range(NRING):
            _s_wait(q)
        return carry

    lax.fori_loop(0, NSUPER, _super, 0)
    plsc.subcore_barrier()

    def _sl(t):
        return pl.ds(row0 + t * ROW_CHUNK, ROW_CHUNK)

    for t in range(NCHUNK):
        q = t % NRING
        if t >= NRING:
            pltpu.make_async_copy(rows_t.at[q], out_hbm.at[c, _sl(t - NRING)],
                                  sem_s.at[q]).wait()
        pltpu.async_copy(acc_sh.at[_sl(t)], rows_t.at[q], sem_g.at[q])
        pltpu.make_async_copy(acc_sh.at[_sl(t)], rows_t.at[q],
                              sem_g.at[q]).wait()
        pltpu.async_copy(rows_t.at[q], out_hbm.at[c, _sl(t)], sem_s.at[q])
    for t in range(NCHUNK - NRING, NCHUNK):
        q = t % NRING
        pltpu.make_async_copy(rows_t.at[q], out_hbm.at[c, _sl(t)],
                              sem_s.at[q]).wait()


_spmm = pl.kernel(
    _spmm_body,
    out_type=jax.ShapeDtypeStruct((NC, NPAD, D), jnp.float32),
    mesh=plsc.VectorSubcoreMesh(core_axis_name="c", subcore_axis_name="s",
                                num_cores=NC, num_subcores=NS),
    scratch_types=[
        pltpu.VMEM((SB, BATCH), jnp.int32),
        pltpu.VMEM((SB, BATCH), jnp.int32),
        pltpu.VMEM((SB, BATCH), jnp.float32),
        pltpu.VMEM((NRING, BATCH, D), jnp.float32),
        pltpu.SemaphoreType.DMA((NRING,)),
        pltpu.SemaphoreType.DMA((NRING,)),
        pltpu.VMEM_SHARED((NPAD, D), jnp.float32),
    ],
)


def kernel(input_feature, adjacency_edge_index, adjacency_edge_weight,
           prior_probability_tensor, W, b):
    support = pl.pallas_call(
        _matmul_body,
        grid=(10,),
        in_specs=[
            pl.BlockSpec((NPAD // 10, D), lambda i: (i, 0)),
            pl.BlockSpec((D, D), lambda i: (0, 0)),
        ],
        out_specs=pl.BlockSpec((NPAD // 10, D), lambda i: (i, 0)),
        out_shape=jax.ShapeDtypeStruct((NPAD, D), jnp.float32),
    )(input_feature, W)

    src = adjacency_edge_index[0].reshape(NC, NS, NSUPER, SB, BATCH)
    dst = adjacency_edge_index[1].reshape(NC, NS, NSUPER, SB, BATCH)
    wgt = adjacency_edge_weight.reshape(NC, NS, NSUPER, SB, BATCH)

    partials = _spmm(support, src, dst, wgt)

    out = pl.pallas_call(
        _epilogue_body,
        grid=(10,),
        in_specs=[
            pl.BlockSpec((1, N // 10, D), lambda i: (0, i, 0)),
            pl.BlockSpec((1, N // 10, D), lambda i: (1, i, 0)),
            pl.BlockSpec((N // 10, D), lambda i: (i, 0)),
            pl.BlockSpec((1, D), lambda i: (0, 0)),
        ],
        out_specs=pl.BlockSpec((N // 10, D), lambda i: (i, 0)),
        out_shape=jax.ShapeDtypeStruct((N, D), jnp.float32),
    )(partials, partials, prior_probability_tensor, b.reshape(1, D))
    return out


# speedup vs baseline: 1.0373x; 1.0373x over previous
import functools

import jax
import jax.numpy as jnp
from jax import lax
from jax.experimental import pallas as pl
from jax.experimental.pallas import tpu as pltpu
from jax.experimental.pallas import tpu_sc as plsc

N = 10000
NPAD = 10240
E = 320000
D = 128

NC = 2
NS = 16
EDGES_PER_WORKER = E // (NC * NS)
BATCH = 80
NBATCH = EDGES_PER_WORKER // BATCH
SB = 25
NSUPER = NBATCH // SB
NRING = 3
ROWS_PER_TILE = NPAD // NS
ROW_CHUNK = BATCH
NCHUNK = ROWS_PER_TILE // ROW_CHUNK


def _matmul_body(x_ref, w_ref, o_ref):
    o_ref[...] = jnp.dot(x_ref[...], w_ref[...],
                         preferred_element_type=jnp.float32)


def _epilogue_body(p0_ref, p1_ref, prior_ref, b_ref, o_ref):
    o_ref[...] = prior_ref[...] * (p0_ref[0] + p1_ref[0]) + b_ref[...]


def _spmm_body(support_hbm, src_hbm, dst_hbm, w_hbm, out_hbm,
               src_t, dst_t, w_t, rows_t, sem_g, sem_s, acc_sh):
    c = lax.axis_index("c")
    s = lax.axis_index("s")
    row0 = s * ROWS_PER_TILE

    zeros16 = jnp.zeros((16,), jnp.float32)

    def _zero_row(r, carry):
        for k in range(D // 16):
            rows_t[0, r, pl.ds(k * 16, 16)] = zeros16
        return carry

    lax.fori_loop(0, ROW_CHUNK, _zero_row, 0)
    for t in range(NCHUNK):
        pltpu.sync_copy(rows_t.at[0], acc_sh.at[pl.ds(row0 + t * ROW_CHUNK,
                                                      ROW_CHUNK)])
    plsc.subcore_barrier()

    def _g_start(j, p):
        pltpu.async_copy(support_hbm.at[src_t.at[j]], rows_t.at[p],
                         sem_g.at[p])

    def _g_wait(j, p):
        pltpu.make_async_copy(support_hbm.at[src_t.at[j]], rows_t.at[p],
                              sem_g.at[p]).wait()

    def _s_start(j, p):
        pltpu.async_copy(rows_t.at[p], acc_sh.at[dst_t.at[j]],
                         sem_s.at[p], add=True)

    def _s_wait(p):
        pltpu.make_async_copy(rows_t.at[p], acc_sh.at[dst_t.at[0]],
                              sem_s.at[p]).wait()

    def _scale(j, p):
        for g in range(BATCH // 16):
            wv16 = w_t[j, pl.ds(g * 16, 16)]

            def _edge(e16, ecarry, wv16=wv16, g=g):
                wv = lax.gather(
                    wv16, jnp.full((16, 1), e16, jnp.int32),
                    dimension_numbers=lax.GatherDimensionNumbers(
                        offset_dims=(), collapsed_slice_dims=(0,),
                        start_index_map=(0,)),
                    slice_sizes=(1,),
                    mode=lax.GatherScatterMode.PROMISE_IN_BOUNDS)
                e = g * 16 + e16
                for k in range(D // 16):
                    sl = pl.ds(k * 16, 16)
                    rows_t[p, e, sl] = rows_t[p, e, sl] * wv
                return ecarry

            lax.fori_loop(0, 16, _edge, 0)

    def _super(t, carry):
        pltpu.sync_copy(src_hbm.at[c, s, t], src_t)
        pltpu.sync_copy(dst_hbm.at[c, s, t], dst_t)
        pltpu.sync_copy(w_hbm.at[c, s, t], w_t)

        _g_start(0, 0)
        _g_start(1, 1)

        def _triple(k, kcarry):
            base = 3 * k
            _g_wait(base, 0)

            @pl.when(k >= 1)
            def _():
                _s_wait(2)

            _g_start(base + 2, 2)
            _scale(base, 0)
            _s_start(base, 0)

            _g_wait(base + 1, 1)
            _s_wait(0)
            _g_start(base + 3, 0)
            _scale(base + 1, 1)
            _s_start(base + 1, 1)

            _g_wait(base + 2, 2)

            @pl.when(k <= 6)
            def _():
                _s_wait(1)
                _g_start(base + 4, 1)

            _scale(base + 2, 2)
            _s_start(base + 2, 2)
            return kcarry

        lax.fori_loop(0, SB // 3, _triple, 0)

        _g_wait(SB - 1, 0)
        _scale(SB - 1, 0)
        _s_start(SB - 1, 0)

        for q in range(NRING):
            _s_wait(q)
        return carry

    lax.fori_loop(0, NSUPER, _super, 0)
    plsc.subcore_barrier()

    def _sl(t):
        return pl.ds(row0 + t * ROW_CHUNK, ROW_CHUNK)

    for t in range(NCHUNK):
        q = t % NRING
        if t >= NRING:
            pltpu.make_async_copy(rows_t.at[q], out_hbm.at[c, _sl(t - NRING)],
                                  sem_s.at[q]).wait()
        pltpu.async_copy(acc_sh.at[_sl(t)], rows_t.at[q], sem_g.at[q])
        pltpu.make_async_copy(acc_sh.at[_sl(t)], rows_t.at[q],
                              sem_g.at[q]).wait()
        pltpu.async_copy(rows_t.at[q], out_hbm.at[c, _sl(t)], sem_s.at[q])
    for t in range(NCHUNK - NRING, NCHUNK):
        q = t % NRING
        pltpu.make_async_copy(rows_t.at[q], out_hbm.at[c, _sl(t)],
                              sem_s.at[q]).wait()


_spmm = pl.kernel(
    _spmm_body,
    out_type=jax.ShapeDtypeStruct((NC, NPAD, D), jnp.float32),
    mesh=plsc.VectorSubcoreMesh(core_axis_name="c", subcore_axis_name="s",
                                num_cores=NC, num_subcores=NS),
    scratch_types=[
        pltpu.VMEM((SB, BATCH), jnp.int32),
        pltpu.VMEM((SB, BATCH), jnp.int32),
        pltpu.VMEM((SB, BATCH), jnp.float32),
        pltpu.VMEM((NRING, BATCH, D), jnp.float32),
        pltpu.SemaphoreType.DMA((NRING,)),
        pltpu.SemaphoreType.DMA((NRING,)),
        pltpu.VMEM_SHARED((NPAD, D), jnp.float32),
    ],
)


def kernel(input_feature, adjacency_edge_index, adjacency_edge_weight,
           prior_probability_tensor, W, b):
    support = pl.pallas_call(
        _matmul_body,
        grid=(4,),
        in_specs=[
            pl.BlockSpec((NPAD // 4, D), lambda i: (i, 0)),
            pl.BlockSpec((D, D), lambda i: (0, 0)),
        ],
        out_specs=pl.BlockSpec((NPAD // 4, D), lambda i: (i, 0)),
        out_shape=jax.ShapeDtypeStruct((NPAD, D), jnp.float32),
    )(input_feature, W)

    src = adjacency_edge_index[0].reshape(NC, NS, NSUPER, SB, BATCH)
    dst = adjacency_edge_index[1].reshape(NC, NS, NSUPER, SB, BATCH)
    wgt = adjacency_edge_weight.reshape(NC, NS, NSUPER, SB, BATCH)

    partials = _spmm(support, src, dst, wgt)

    out = pl.pallas_call(
        _epilogue_body,
        grid=(5,),
        in_specs=[
            pl.BlockSpec((1, N // 5, D), lambda i: (0, i, 0)),
            pl.BlockSpec((1, N // 5, D), lambda i: (1, i, 0)),
            pl.BlockSpec((N // 5, D), lambda i: (i, 0)),
            pl.BlockSpec((1, D), lambda i: (0, 0)),
        ],
        out_specs=pl.BlockSpec((N // 5, D), lambda i: (i, 0)),
        out_shape=jax.ShapeDtypeStruct((N, D), jnp.float32),
    )(partials, partials, prior_probability_tensor, b.reshape(1, D))
    return out


# speedup vs baseline: 1.0482x; 1.0105x over previous
import functools

import jax
import jax.numpy as jnp
from jax import lax
from jax.experimental import pallas as pl
from jax.experimental.pallas import tpu as pltpu
from jax.experimental.pallas import tpu_sc as plsc

N = 10000
NPAD = 10240
E = 320000
D = 128

NC = 2
NS = 16
EDGES_PER_WORKER = E // (NC * NS)
BATCH = 80
NBATCH = EDGES_PER_WORKER // BATCH
SB = 25
NSUPER = NBATCH // SB
NRING = 3
ROWS_PER_TILE = NPAD // NS
ROW_CHUNK = BATCH
NCHUNK = ROWS_PER_TILE // ROW_CHUNK


def _matmul_body(x_ref, w_ref, o_ref):
    o_ref[...] = jnp.dot(x_ref[...], w_ref[...],
                         preferred_element_type=jnp.float32)


def _epilogue_body(p0_ref, p1_ref, prior_ref, b_ref, o_ref):
    o_ref[...] = prior_ref[...] * (p0_ref[0] + p1_ref[0]) + b_ref[...]


def _spmm_body(support_hbm, src_hbm, dst_hbm, w_hbm, out_hbm,
               src_t, dst_t, w_t, rows_t, sem_g, sem_s, acc_sh):
    c = lax.axis_index("c")
    s = lax.axis_index("s")
    row0 = s * ROWS_PER_TILE

    zeros16 = jnp.zeros((16,), jnp.float32)

    def _zero_row(r, carry):
        for k in range(D // 16):
            rows_t[0, r, pl.ds(k * 16, 16)] = zeros16
        return carry

    lax.fori_loop(0, ROW_CHUNK, _zero_row, 0)
    for t in range(NCHUNK):
        pltpu.sync_copy(rows_t.at[0], acc_sh.at[pl.ds(row0 + t * ROW_CHUNK,
                                                      ROW_CHUNK)])
    plsc.subcore_barrier()

    def _g_start(j, p):
        pltpu.async_copy(support_hbm.at[src_t.at[j]], rows_t.at[p],
                         sem_g.at[p])

    def _g_wait(j, p):
        pltpu.make_async_copy(support_hbm.at[src_t.at[j]], rows_t.at[p],
                              sem_g.at[p]).wait()

    def _s_start(j, p):
        pltpu.async_copy(rows_t.at[p], acc_sh.at[dst_t.at[j]],
                         sem_s.at[p], add=True)

    def _s_wait(p):
        pltpu.make_async_copy(rows_t.at[p], acc_sh.at[dst_t.at[0]],
                              sem_s.at[p]).wait()

    def _scale(j, p):
        for g in range(BATCH // 16):
            wv16 = w_t[j, pl.ds(g * 16, 16)]

            def _edge(e16, ecarry, wv16=wv16, g=g):
                wv = lax.gather(
                    wv16, jnp.full((16, 1), e16, jnp.int32),
                    dimension_numbers=lax.GatherDimensionNumbers(
                        offset_dims=(), collapsed_slice_dims=(0,),
                        start_index_map=(0,)),
                    slice_sizes=(1,),
                    mode=lax.GatherScatterMode.PROMISE_IN_BOUNDS)
                e = g * 16 + e16
                for k in range(D // 16):
                    sl = pl.ds(k * 16, 16)
                    rows_t[p, e, sl] = rows_t[p, e, sl] * wv
                return ecarry

            lax.fori_loop(0, 16, _edge, 0)

    def _super(t, carry):
        pltpu.sync_copy(src_hbm.at[c, s, t], src_t)
        pltpu.sync_copy(dst_hbm.at[c, s, t], dst_t)
        pltpu.sync_copy(w_hbm.at[c, s, t], w_t)

        _g_start(0, 0)
        _g_start(1, 1)

        def _triple(k, kcarry):
            base = 3 * k
            _g_wait(base, 0)

            @pl.when(k >= 1)
            def _():
                _s_wait(2)

            _g_start(base + 2, 2)
            _scale(base, 0)
            _s_start(base, 0)

            _g_wait(base + 1, 1)
            _s_wait(0)
            _g_start(base + 3, 0)
            _scale(base + 1, 1)
            _s_start(base + 1, 1)

            _g_wait(base + 2, 2)

            @pl.when(k <= 6)
            def _():
                _s_wait(1)
                _g_start(base + 4, 1)

            _scale(base + 2, 2)
            _s_start(base + 2, 2)
            return kcarry

        lax.fori_loop(0, SB // 3, _triple, 0)

        _g_wait(SB - 1, 0)
        _scale(SB - 1, 0)
        _s_start(SB - 1, 0)

        for q in range(NRING):
            _s_wait(q)
        return carry

    lax.fori_loop(0, NSUPER, _super, 0)
    plsc.subcore_barrier()

    def _sl(t):
        return pl.ds(row0 + t * ROW_CHUNK, ROW_CHUNK)

    for t in range(NCHUNK):
        q = t % NRING
        if t >= NRING:
            pltpu.make_async_copy(rows_t.at[q], out_hbm.at[c, _sl(t - NRING)],
                                  sem_s.at[q]).wait()
        pltpu.async_copy(acc_sh.at[_sl(t)], rows_t.at[q], sem_g.at[q])
        pltpu.make_async_copy(acc_sh.at[_sl(t)], rows_t.at[q],
                              sem_g.at[q]).wait()
        pltpu.async_copy(rows_t.at[q], out_hbm.at[c, _sl(t)], sem_s.at[q])
    for t in range(NCHUNK - NRING, NCHUNK):
        q = t % NRING
        pltpu.make_async_copy(rows_t.at[q], out_hbm.at[c, _sl(t)],
                              sem_s.at[q]).wait()


_spmm = pl.kernel(
    _spmm_body,
    out_type=jax.ShapeDtypeStruct((NC, NPAD, D), jnp.float32),
    mesh=plsc.VectorSubcoreMesh(core_axis_name="c", subcore_axis_name="s",
                                num_cores=NC, num_subcores=NS),
    scratch_types=[
        pltpu.VMEM((SB, BATCH), jnp.int32),
        pltpu.VMEM((SB, BATCH), jnp.int32),
        pltpu.VMEM((SB, BATCH), jnp.float32),
        pltpu.VMEM((NRING, BATCH, D), jnp.float32),
        pltpu.SemaphoreType.DMA((NRING,)),
        pltpu.SemaphoreType.DMA((NRING,)),
        pltpu.VMEM_SHARED((NPAD, D), jnp.float32),
    ],
)


def kernel(input_feature, adjacency_edge_index, adjacency_edge_weight,
           prior_probability_tensor, W, b):
    support = pl.pallas_call(
        _matmul_body,
        grid=(2,),
        in_specs=[
            pl.BlockSpec((NPAD // 2, D), lambda i: (i, 0)),
            pl.BlockSpec((D, D), lambda i: (0, 0)),
        ],
        out_specs=pl.BlockSpec((NPAD // 2, D), lambda i: (i, 0)),
        out_shape=jax.ShapeDtypeStruct((NPAD, D), jnp.float32),
    )(input_feature, W)

    src = adjacency_edge_index[0].reshape(NC, NS, NSUPER, SB, BATCH)
    dst = adjacency_edge_index[1].reshape(NC, NS, NSUPER, SB, BATCH)
    wgt = adjacency_edge_weight.reshape(NC, NS, NSUPER, SB, BATCH)

    partials = _spmm(support, src, dst, wgt)

    out = pl.pallas_call(
        _epilogue_body,
        grid=(2,),
        in_specs=[
            pl.BlockSpec((1, N // 2, D), lambda i: (0, i, 0)),
            pl.BlockSpec((1, N // 2, D), lambda i: (1, i, 0)),
            pl.BlockSpec((N // 2, D), lambda i: (i, 0)),
            pl.BlockSpec((1, D), lambda i: (0, 0)),
        ],
        out_specs=pl.BlockSpec((N // 2, D), lambda i: (i, 0)),
        out_shape=jax.ShapeDtypeStruct((N, D), jnp.float32),
    )(partials, partials, prior_probability_tensor, b.reshape(1, D))
    return out


# speedup vs baseline: 1.0858x; 1.0359x over previous
import functools

import jax
import jax.numpy as jnp
from jax import lax
from jax.experimental import pallas as pl
from jax.experimental.pallas import tpu as pltpu
from jax.experimental.pallas import tpu_sc as plsc

N = 10000
NPAD = 10240
E = 320000
D = 128

NC = 2
NS = 16
EDGES_PER_WORKER = E // (NC * NS)
BATCH = 80
NBATCH = EDGES_PER_WORKER // BATCH
SB = 25
NSUPER = NBATCH // SB
NRING = 3
ROWS_PER_TILE = NPAD // NS
ROW_CHUNK = BATCH
NCHUNK = ROWS_PER_TILE // ROW_CHUNK


def _matmul_body(x_ref, w_ref, o_ref):
    o_ref[...] = jnp.dot(x_ref[...], w_ref[...],
                         preferred_element_type=jnp.float32)


def _epilogue_body(p0_ref, p1_ref, prior_ref, b_ref, o_ref):
    o_ref[...] = prior_ref[...] * (p0_ref[0] + p1_ref[0]) + b_ref[...]


def _spmm_body(support_hbm, src_hbm, dst_hbm, w_hbm, out_hbm,
               src_t, dst_t, w_t, rows_t, sem_g, sem_s, acc_sh):
    c = lax.axis_index("c")
    s = lax.axis_index("s")
    row0 = s * ROWS_PER_TILE

    zeros16 = jnp.zeros((16,), jnp.float32)

    def _zero_row(r, carry):
        for k in range(D // 16):
            rows_t[0, r, pl.ds(k * 16, 16)] = zeros16
        return carry

    lax.fori_loop(0, ROW_CHUNK, _zero_row, 0)

    def _zsem(t):
        q = t % (2 * NRING)
        return sem_g.at[q] if q < NRING else sem_s.at[q - NRING]

    def _zdst(t):
        return acc_sh.at[pl.ds(row0 + t * ROW_CHUNK, ROW_CHUNK)]

    for t in range(NCHUNK):
        if t >= 2 * NRING:
            pltpu.make_async_copy(rows_t.at[0], _zdst(t - 2 * NRING),
                                  _zsem(t)).wait()
        pltpu.async_copy(rows_t.at[0], _zdst(t), _zsem(t))
    for t in range(max(0, NCHUNK - 2 * NRING), NCHUNK):
        pltpu.make_async_copy(rows_t.at[0], _zdst(t), _zsem(t)).wait()
    plsc.subcore_barrier()

    def _g_start(j, p):
        pltpu.async_copy(support_hbm.at[src_t.at[j]], rows_t.at[p],
                         sem_g.at[p])

    def _g_wait(j, p):
        pltpu.make_async_copy(support_hbm.at[src_t.at[j]], rows_t.at[p],
                              sem_g.at[p]).wait()

    def _s_start(j, p):
        pltpu.async_copy(rows_t.at[p], acc_sh.at[dst_t.at[j]],
                         sem_s.at[p], add=True)

    def _s_wait(p):
        pltpu.make_async_copy(rows_t.at[p], acc_sh.at[dst_t.at[0]],
                              sem_s.at[p]).wait()

    def _scale(j, p):
        for g in range(BATCH // 16):
            wv16 = w_t[j, pl.ds(g * 16, 16)]

            def _edge(e16, ecarry, wv16=wv16, g=g):
                wv = lax.gather(
                    wv16, jnp.full((16, 1), e16, jnp.int32),
                    dimension_numbers=lax.GatherDimensionNumbers(
                        offset_dims=(), collapsed_slice_dims=(0,),
                        start_index_map=(0,)),
                    slice_sizes=(1,),
                    mode=lax.GatherScatterMode.PROMISE_IN_BOUNDS)
                e = g * 16 + e16
                for k in range(D // 16):
                    sl = pl.ds(k * 16, 16)
                    rows_t[p, e, sl] = rows_t[p, e, sl] * wv
                return ecarry

            lax.fori_loop(0, 16, _edge, 0)

    def _super(t, carry):
        pltpu.async_copy(src_hbm.at[c, s, t], src_t, sem_g.at[0])
        pltpu.async_copy(dst_hbm.at[c, s, t], dst_t, sem_g.at[1])
        pltpu.async_copy(w_hbm.at[c, s, t], w_t, sem_g.at[2])
        pltpu.make_async_copy(src_hbm.at[c, s, t], src_t, sem_g.at[0]).wait()
        pltpu.make_async_copy(dst_hbm.at[c, s, t], dst_t, sem_g.at[1]).wait()
        pltpu.make_async_copy(w_hbm.at[c, s, t], w_t, sem_g.at[2]).wait()

        _g_start(0, 0)
        _g_start(1, 1)

        def _triple(k, kcarry):
            base = 3 * k
            _g_wait(base, 0)

            @pl.when(k >= 1)
            def _():
                _s_wait(2)

            _g_start(base + 2, 2)
            _scale(base, 0)
            _s_start(base, 0)

            _g_wait(base + 1, 1)
            _s_wait(0)
            _g_start(base + 3, 0)
            _scale(base + 1, 1)
            _s_start(base + 1, 1)

            _g_wait(base + 2, 2)

            @pl.when(k <= 6)
            def _():
                _s_wait(1)
                _g_start(base + 4, 1)

            _scale(base + 2, 2)
            _s_start(base + 2, 2)
            return kcarry

        lax.fori_loop(0, SB // 3, _triple, 0)

        _g_wait(SB - 1, 0)
        _scale(SB - 1, 0)
        _s_start(SB - 1, 0)

        for q in range(NRING):
            _s_wait(q)
        return carry

    lax.fori_loop(0, NSUPER, _super, 0)
    plsc.subcore_barrier()

    def _sl(t):
        return pl.ds(row0 + t * ROW_CHUNK, ROW_CHUNK)

    for t in range(NCHUNK):
        q = t % NRING
        if t >= NRING:
            pltpu.make_async_copy(rows_t.at[q], out_hbm.at[c, _sl(t - NRING)],
                                  sem_s.at[q]).wait()
        pltpu.async_copy(acc_sh.at[_sl(t)], rows_t.at[q], sem_g.at[q])
        pltpu.make_async_copy(acc_sh.at[_sl(t)], rows_t.at[q],
                              sem_g.at[q]).wait()
        pltpu.async_copy(rows_t.at[q], out_hbm.at[c, _sl(t)], sem_s.at[q])
    for t in range(NCHUNK - NRING, NCHUNK):
        q = t % NRING
        pltpu.make_async_copy(rows_t.at[q], out_hbm.at[c, _sl(t)],
                              sem_s.at[q]).wait()


_spmm = pl.kernel(
    _spmm_body,
    out_type=jax.ShapeDtypeStruct((NC, NPAD, D), jnp.float32),
    mesh=plsc.VectorSubcoreMesh(core_axis_name="c", subcore_axis_name="s",
                                num_cores=NC, num_subcores=NS),
    scratch_types=[
        pltpu.VMEM((SB, BATCH), jnp.int32),
        pltpu.VMEM((SB, BATCH), jnp.int32),
        pltpu.VMEM((SB, BATCH), jnp.float32),
        pltpu.VMEM((NRING, BATCH, D), jnp.float32),
        pltpu.SemaphoreType.DMA((NRING,)),
        pltpu.SemaphoreType.DMA((NRING,)),
        pltpu.VMEM_SHARED((NPAD, D), jnp.float32),
    ],
)


def kernel(input_feature, adjacency_edge_index, adjacency_edge_weight,
           prior_probability_tensor, W, b):
    support = pl.pallas_call(
        _matmul_body,
        grid=(2,),
        in_specs=[
            pl.BlockSpec((NPAD // 2, D), lambda i: (i, 0)),
            pl.BlockSpec((D, D), lambda i: (0, 0)),
        ],
        out_specs=pl.BlockSpec((NPAD // 2, D), lambda i: (i, 0)),
        out_shape=jax.ShapeDtypeStruct((NPAD, D), jnp.float32),
    )(input_feature, W)

    src = adjacency_edge_index[0].reshape(NC, NS, NSUPER, SB, BATCH)
    dst = adjacency_edge_index[1].reshape(NC, NS, NSUPER, SB, BATCH)
    wgt = adjacency_edge_weight.reshape(NC, NS, NSUPER, SB, BATCH)

    partials = _spmm(support, src, dst, wgt)

    out = pl.pallas_call(
        _epilogue_body,
        grid=(2,),
        in_specs=[
            pl.BlockSpec((1, N // 2, D), lambda i: (0, i, 0)),
            pl.BlockSpec((1, N // 2, D), lambda i: (1, i, 0)),
            pl.BlockSpec((N // 2, D), lambda i: (i, 0)),
            pl.BlockSpec((1, D), lambda i: (0, 0)),
        ],
        out_specs=pl.BlockSpec((N // 2, D), lambda i: (i, 0)),
        out_shape=jax.ShapeDtypeStruct((N, D), jnp.float32),
    )(partials, partials, prior_probability_tensor, b.reshape(1, D))
    return out
